# Initial kernel scaffold; baseline (speedup 1.0000x reference)
import jax
import jax.numpy as jnp
from jax.experimental import pallas as pl


def kernel(inputs):
    raise NotImplementedError("write your pallas kernel here")


# speedup vs baseline: 8.0829x; 8.0829x over previous
import functools

import jax
import jax.numpy as jnp
from jax import lax
from jax.experimental import pallas as pl
from jax.experimental.pallas import tpu as pltpu
from jax.experimental.pallas import tpu_sc as plsc

K = 256
B = 128
N = 32768
L = 16
SHIFT = 20
NB = 1 << (31 - SHIFT)
NVEC = N // L
NBLK = NB // L
NWORKERS = 32
ROWS_PER_W = B // NWORKERS


def _topk_body(x_hbm, out_hbm, row_v, hist_v, cand_v):
    cid = lax.axis_index("c")
    sid = lax.axis_index("s")
    wid = sid * 2 + cid

    lane = lax.broadcasted_iota(jnp.int32, (L,), 0)
    lane_off = lane * NB
    ones = jnp.ones((L,), jnp.int32)
    zeros = jnp.zeros((L,), jnp.int32)
    mask31 = jnp.int32(0x7FFFFFFF)

    def process_row(j, _carry):
        r = wid * ROWS_PER_W + j
        pltpu.sync_copy(x_hbm.at[r], row_v)

        def zbody(i, _):
            hist_v[pl.ds(i * L, L)] = zeros
            return 0
        lax.fori_loop(0, NB * L // L, zbody, 0)

        def hbody(i, _):
            v = row_v[pl.ds(i * L, L)]
            a = lax.bitwise_and(v, mask31)
            row_v[pl.ds(i * L, L)] = a
            bucket = lax.shift_right_logical(a, SHIFT)
            plsc.addupdate_scatter(hist_v, [lane_off + bucket], ones)
            return 0
        lax.fori_loop(0, NVEC, hbody, 0)

        def sbody(i, carry):
            cum, found, bkt, kin = carry
            blk = NBLK - 1 - i
            acc = hist_v[pl.ds(blk * L, L)]
            for l in range(1, L):
                acc = acc + hist_v[pl.ds(l * NB + blk * L, L)]
            bs = jnp.sum(acc)
            need = K - cum
            racc = lax.rev(acc, (0,))
            csum = plsc.cumsum(racc)
            maskv = csum >= need
            jj = jnp.max(plsc.all_reduce_ffs(maskv))
            csum_at = jnp.sum(jnp.where(lane == jj, csum, 0))
            cnt_b = jnp.sum(jnp.where(lane == jj, racc, 0))
            hit = jnp.logical_and(found == 0, bs >= need)
            bkt = jnp.where(hit, blk * L + (L - 1) - jj, bkt)
            kin = jnp.where(hit, need - (csum_at - cnt_b), kin)
            found = jnp.where(hit, 1, found)
            return (cum + bs, found, bkt, kin)
        _, _, bkt, kin = lax.fori_loop(
            0, NBLK, sbody,
            (jnp.int32(0), jnp.int32(0), jnp.int32(0), jnp.int32(0)))

        def ebody(i, c):
            a = row_v[pl.ds(i * L, L)]
            m = lax.shift_right_logical(a, SHIFT) == bkt
            mi = m.astype(jnp.int32)
            pos = c + plsc.cumsum(mi) - 1
            plsc.store_scatter(cand_v, [pos], a, mask=m)
            return c + jnp.sum(mi)
        c = lax.fori_loop(0, NVEC, ebody, jnp.int32(0))
        cand_v[pl.ds(c, L)] = zeros

        base = bkt << SHIFT
        nv = (c + L - 1) // L
        lo = jnp.int32(0)
        for bit in range(SHIFT - 1, -1, -1):
            trial = base | lo | (1 << bit)
            def cbody(i, cnt, trial=trial):
                v = cand_v[pl.ds(i * L, L)]
                return cnt + jnp.sum((v >= trial).astype(jnp.int32))
            cnt = lax.fori_loop(0, nv, cbody, jnp.int32(0))
            lo = jnp.where(cnt >= kin, lo | (1 << bit), lo)
        T = base | lo

        def obody(i, _):
            a = row_v[pl.ds(i * L, L)]
            row_v[pl.ds(i * L, L)] = jnp.where(a >= T, a, 0)
            return 0
        lax.fori_loop(0, NVEC, obody, 0)
        pltpu.sync_copy(row_v, out_hbm.at[r])
        return 0

    lax.fori_loop(0, ROWS_PER_W, process_row, 0)


@functools.partial(jax.jit, static_argnums=())
def _topk_mask(bits):
    mesh = plsc.VectorSubcoreMesh(core_axis_name="c", subcore_axis_name="s")
    f = functools.partial(
        pl.kernel,
        out_type=jax.ShapeDtypeStruct((B, N), jnp.int32),
        mesh=mesh,
        scratch_types=[
            pltpu.VMEM((N,), jnp.int32),
            pltpu.VMEM((NB * L,), jnp.int32),
            pltpu.VMEM((N + L,), jnp.int32),
        ],
        compiler_params=pltpu.CompilerParams(needs_layout_passes=False),
    )(_topk_body)
    return f(bits)


def kernel(inputs):
    bits = lax.bitcast_convert_type(inputs, jnp.int32)
    out_bits = _topk_mask(bits)
    return lax.bitcast_convert_type(out_bits, jnp.float32)
